# baseline (speedup 1.0000x reference)
import functools

import jax
import jax.numpy as jnp
import numpy as np
from jax import lax
from jax.experimental import pallas as pl
from jax.experimental.pallas import tpu as pltpu
from jax.experimental.pallas import tpu_sc as plsc

EMB_DIM = 128
SEQ = 200
SCALE = float(np.sqrt(float(EMB_DIM)))

NUM_CORES = 2
NUM_SUBCORES = 16
NUM_WORKERS = NUM_CORES * NUM_SUBCORES
CHUNK = 256
VEC = 16


def _pos_table(max_length, emb_dim):
    pos = np.arange(max_length)[:, np.newaxis]
    div_term = np.exp(np.arange(0, emb_dim, 2) * -(np.log(10000.0) / emb_dim))
    pos_emb = pos * div_term
    pos_emb = np.stack([np.sin(pos_emb), np.cos(pos_emb)], axis=1).reshape(max_length, -1)
    pos_emb[1:, 1::2] = 0
    return pos_emb.astype(np.float32)


_POS = _pos_table(SEQ + 1, EMB_DIM)[:SEQ]


def _encode_body(idx_hbm, sidx_hbm, emb_hbm, seg_hbm, pos_hbm, out_hbm,
                 idx_v, sidx_v, rows_v, segrows_v, pos_v, sem_e, sem_s,
                 rows_per_w, n_chunks):
    wid = lax.axis_index("s") * NUM_CORES + lax.axis_index("c")
    wbase = wid * rows_per_w

    pltpu.sync_copy(pos_hbm, pos_v)

    def chunk_body(ci, carry):
        base = wbase + ci * CHUNK
        pltpu.sync_copy(idx_hbm.at[pl.ds(base, CHUNK)], idx_v)
        pltpu.sync_copy(sidx_hbm.at[pl.ds(base, CHUNK)], sidx_v)
        ge = pltpu.async_copy(emb_hbm.at[idx_v], rows_v, sem_e)
        gs = pltpu.async_copy(seg_hbm.at[sidx_v], segrows_v, sem_s)
        ge.wait()
        gs.wait()

        def row_body(r, carry2):
            l = lax.rem(base + r, SEQ)
            for g in range(EMB_DIM // VEC):
                sl = pl.ds(g * VEC, VEC)
                e = rows_v[r, sl]
                p = pos_v[l, sl]
                sgr = segrows_v[r, sl]
                rows_v[r, sl] = e * SCALE + p + sgr
            return carry2

        lax.fori_loop(0, CHUNK, row_body, 0, unroll=False)
        pltpu.sync_copy(rows_v, out_hbm.at[pl.ds(base, CHUNK)])
        return carry

    lax.fori_loop(0, n_chunks, chunk_body, 0, unroll=False)


def kernel(batch, segment_ids, emb_table, seg_table):
    B, L = batch.shape
    N = B * L
    rows_per_w = N // NUM_WORKERS
    n_chunks = rows_per_w // CHUNK

    idx = batch.reshape(N).astype(jnp.int32)
    sidx = segment_ids.reshape(N).astype(jnp.int32)
    pos = jnp.asarray(_POS)

    body = functools.partial(_encode_body, rows_per_w=rows_per_w, n_chunks=n_chunks)
    run = pl.kernel(
        body,
        out_type=jax.ShapeDtypeStruct((N, EMB_DIM), jnp.float32),
        mesh=plsc.VectorSubcoreMesh(
            core_axis_name="c", subcore_axis_name="s",
            num_cores=NUM_CORES, num_subcores=NUM_SUBCORES),
        scratch_types=[
            pltpu.VMEM((CHUNK,), jnp.int32),
            pltpu.VMEM((CHUNK,), jnp.int32),
            pltpu.VMEM((CHUNK, EMB_DIM), jnp.float32),
            pltpu.VMEM((CHUNK, EMB_DIM), jnp.float32),
            pltpu.VMEM((SEQ, EMB_DIM), jnp.float32),
            pltpu.SemaphoreType.DMA,
            pltpu.SemaphoreType.DMA,
        ],
    )
    out = run(idx, sidx, emb_table, seg_table, pos)
    return out.reshape(B, L, EMB_DIM)


# speedup vs baseline: 1.0010x; 1.0010x over previous
import functools

import jax
import jax.numpy as jnp
import numpy as np
from jax import lax
from jax.experimental import pallas as pl
from jax.experimental.pallas import tpu as pltpu
from jax.experimental.pallas import tpu_sc as plsc

EMB_DIM = 128
SEQ = 200
SCALE = float(np.sqrt(float(EMB_DIM)))

NUM_CORES = 2
NUM_SUBCORES = 16
NUM_WORKERS = NUM_CORES * NUM_SUBCORES
CHUNK = 80
VEC = 16
GRPS = EMB_DIM // VEC


def _pos_table(max_length, emb_dim):
    pos = np.arange(max_length)[:, np.newaxis]
    div_term = np.exp(np.arange(0, emb_dim, 2) * -(np.log(10000.0) / emb_dim))
    pos_emb = pos * div_term
    pos_emb = np.stack([np.sin(pos_emb), np.cos(pos_emb)], axis=1).reshape(max_length, -1)
    pos_emb[1:, 1::2] = 0
    return pos_emb.astype(np.float32)


_POS = _pos_table(SEQ + 1, EMB_DIM)[:SEQ]


def _encode_body(idx_hbm, sidx_hbm, emb_hbm, seg_hbm, pos_hbm, out_hbm,
                 idx_all, sidx_all, pos_v, rows, segrows, obuf, gsem, ssem, osem,
                 rows_per_w, n_chunks):
    wid = lax.axis_index("s") * NUM_CORES + lax.axis_index("c")
    wbase = wid * rows_per_w

    pltpu.sync_copy(idx_hbm.at[pl.ds(wbase, rows_per_w)], idx_all)
    pltpu.sync_copy(sidx_hbm.at[pl.ds(wbase, rows_per_w)], sidx_all)
    pltpu.sync_copy(pos_hbm, pos_v)

    def issue(s, ci):
        off = ci * CHUNK
        pltpu.async_copy(emb_hbm.at[idx_all.at[pl.ds(off, CHUNK)]], rows[s], gsem[s])
        pltpu.async_copy(seg_hbm.at[sidx_all.at[pl.ds(off, CHUNK)]], segrows[s], ssem[s])

    def wait_gathers(s, ci):
        off = ci * CHUNK
        pltpu.make_async_copy(emb_hbm.at[idx_all.at[pl.ds(off, CHUNK)]], rows[s], gsem[s]).wait()
        pltpu.make_async_copy(seg_hbm.at[sidx_all.at[pl.ds(off, CHUNK)]], segrows[s], ssem[s]).wait()

    def out_start(s, ci):
        base = wbase + ci * CHUNK
        pltpu.async_copy(obuf[s], out_hbm.at[pl.ds(base, CHUNK)], osem[s])

    def out_wait(s, ci):
        base = wbase + ci * CHUNK
        pltpu.make_async_copy(obuf[s], out_hbm.at[pl.ds(base, CHUNK)], osem[s]).wait()

    def compute(s, ci):
        cbase = ci * CHUNK

        @plsc.parallel_loop(0, CHUNK, unroll=2)
        def _(r):
            l = lax.rem(cbase + r, SEQ)
            for g in range(GRPS):
                sl = pl.ds(g * VEC, VEC)
                obuf[s][r, sl] = rows[s][r, sl] * SCALE + pos_v[l, sl] + segrows[s][r, sl]

    issue(0, 0)
    issue(1, 1)

    def chunk_pair(ci2, carry):
        for s in (0, 1):
            ci = ci2 * 2 + s
            wait_gathers(s, ci)

            @pl.when(ci2 >= 1)
            def _():
                out_wait(s, ci - 2)

            compute(s, ci)

            @pl.when(ci + 2 < n_chunks)
            def _():
                issue(s, ci + 2)

            out_start(s, ci)
        return carry

    lax.fori_loop(0, n_chunks // 2, chunk_pair, 0, unroll=False)
    out_wait(0, n_chunks - 2)
    out_wait(1, n_chunks - 1)


def kernel(batch, segment_ids, emb_table, seg_table):
    B, L = batch.shape
    N = B * L
    rows_per_w = N // NUM_WORKERS
    n_chunks = rows_per_w // CHUNK

    idx = batch.reshape(N).astype(jnp.int32)
    sidx = segment_ids.reshape(N).astype(jnp.int32)
    pos = jnp.asarray(_POS)

    body = functools.partial(_encode_body, rows_per_w=rows_per_w, n_chunks=n_chunks)
    run = pl.kernel(
        body,
        out_type=jax.ShapeDtypeStruct((N, EMB_DIM), jnp.float32),
        mesh=plsc.VectorSubcoreMesh(
            core_axis_name="c", subcore_axis_name="s",
            num_cores=NUM_CORES, num_subcores=NUM_SUBCORES),
        scratch_types=[
            pltpu.VMEM((rows_per_w,), jnp.int32),
            pltpu.VMEM((rows_per_w,), jnp.int32),
            pltpu.VMEM((SEQ, EMB_DIM), jnp.float32),
            [pltpu.VMEM((CHUNK, EMB_DIM), jnp.float32) for _ in range(2)],
            [pltpu.VMEM((CHUNK, EMB_DIM), jnp.float32) for _ in range(2)],
            [pltpu.VMEM((CHUNK, EMB_DIM), jnp.float32) for _ in range(2)],
            [pltpu.SemaphoreType.DMA for _ in range(2)],
            [pltpu.SemaphoreType.DMA for _ in range(2)],
            [pltpu.SemaphoreType.DMA for _ in range(2)],
        ],
    )
    out = run(idx, sidx, emb_table, seg_table, pos)
    return out.reshape(B, L, EMB_DIM)


# speedup vs baseline: 11.0030x; 10.9915x over previous
import functools

import jax
import jax.numpy as jnp
import numpy as np
from jax import lax
from jax.experimental import pallas as pl
from jax.experimental.pallas import tpu as pltpu
from jax.experimental.pallas import tpu_sc as plsc

EMB_DIM = 128
SEQ = 200
SCALE = float(np.sqrt(float(EMB_DIM)))

NUM_CORES = 2
NUM_SUBCORES = 16
NUM_WORKERS = NUM_CORES * NUM_SUBCORES
CHUNK = 80
VEC = 16
GRPS = EMB_DIM // VEC


def _pos_table(max_length, emb_dim):
    pos = np.arange(max_length)[:, np.newaxis]
    div_term = np.exp(np.arange(0, emb_dim, 2) * -(np.log(10000.0) / emb_dim))
    pos_emb = pos * div_term
    pos_emb = np.stack([np.sin(pos_emb), np.cos(pos_emb)], axis=1).reshape(max_length, -1)
    pos_emb[1:, 1::2] = 0
    return pos_emb.astype(np.float32)


_POS = _pos_table(SEQ + 1, EMB_DIM)[:SEQ]


def _encode_body(idx_hbm, sidx_hbm, emb_hbm, seg_hbm, pos_hbm, out_hbm,
                 idx_all, sidx_all, aidx_all, comb, segv, rows, obuf, gsem, osem,
                 rows_per_w, n_chunks):
    wid = lax.axis_index("s") * NUM_CORES + lax.axis_index("c")
    wbase = wid * rows_per_w

    pltpu.sync_copy(idx_hbm.at[pl.ds(wbase, rows_per_w)], idx_all)
    pltpu.sync_copy(sidx_hbm.at[pl.ds(wbase, rows_per_w)], sidx_all)
    pltpu.sync_copy(pos_hbm, comb.at[pl.ds(0, SEQ)])
    pltpu.sync_copy(pos_hbm, comb.at[pl.ds(SEQ, SEQ)])
    pltpu.sync_copy(seg_hbm, segv)

    @plsc.parallel_loop(0, SEQ, unroll=2)
    def _(r):
        for g in range(GRPS):
            sl = pl.ds(g * VEC, VEC)
            comb[r, sl] = comb[r, sl] + segv[0, sl]
            comb[SEQ + r, sl] = comb[SEQ + r, sl] + segv[1, sl]

    @plsc.parallel_loop(0, rows_per_w // VEC, unroll=2)
    def _(v):
        base = v * VEC
        l16 = lax.rem(base + lax.iota(jnp.int32, VEC), SEQ)
        aidx_all[pl.ds(base, VEC)] = sidx_all[pl.ds(base, VEC)] * SEQ + l16

    def issue(s, ci):
        off = ci * CHUNK
        pltpu.async_copy(emb_hbm.at[idx_all.at[pl.ds(off, CHUNK)]], rows[s], gsem[s])

    def wait_gather(s, ci):
        off = ci * CHUNK
        pltpu.make_async_copy(emb_hbm.at[idx_all.at[pl.ds(off, CHUNK)]], rows[s], gsem[s]).wait()

    def out_start(s, ci):
        base = wbase + ci * CHUNK
        pltpu.async_copy(obuf[s], out_hbm.at[pl.ds(base, CHUNK)], osem[s])

    def out_wait(s, ci):
        base = wbase + ci * CHUNK
        pltpu.make_async_copy(obuf[s], out_hbm.at[pl.ds(base, CHUNK)], osem[s]).wait()

    def compute(s, ci):
        cbase = ci * CHUNK

        @plsc.parallel_loop(0, CHUNK // VEC)
        def _(v):
            r0 = v * VEC
            ar16 = aidx_all[pl.ds(cbase + r0, VEC)]
            for j in range(VEC):
                ar = ar16[j]
                for g in range(GRPS):
                    sl = pl.ds(g * VEC, VEC)
                    obuf[s][r0 + j, sl] = rows[s][r0 + j, sl] * SCALE + comb[ar, sl]

    issue(0, 0)
    issue(1, 1)

    def chunk_pair(ci2, carry):
        for s in (0, 1):
            ci = ci2 * 2 + s
            wait_gather(s, ci)

            @pl.when(ci2 >= 1)
            def _():
                out_wait(s, ci - 2)

            compute(s, ci)

            @pl.when(ci + 2 < n_chunks)
            def _():
                issue(s, ci + 2)

            out_start(s, ci)
        return carry

    lax.fori_loop(0, n_chunks // 2, chunk_pair, 0, unroll=False)
    out_wait(0, n_chunks - 2)
    out_wait(1, n_chunks - 1)


def kernel(batch, segment_ids, emb_table, seg_table):
    B, L = batch.shape
    N = B * L
    rows_per_w = N // NUM_WORKERS
    n_chunks = rows_per_w // CHUNK

    idx = batch.reshape(N).astype(jnp.int32)
    sidx = segment_ids.reshape(N).astype(jnp.int32)
    pos = jnp.asarray(_POS)

    body = functools.partial(_encode_body, rows_per_w=rows_per_w, n_chunks=n_chunks)
    run = pl.kernel(
        body,
        out_type=jax.ShapeDtypeStruct((N, EMB_DIM), jnp.float32),
        mesh=plsc.VectorSubcoreMesh(
            core_axis_name="c", subcore_axis_name="s",
            num_cores=NUM_CORES, num_subcores=NUM_SUBCORES),
        scratch_types=[
            pltpu.VMEM((rows_per_w,), jnp.int32),
            pltpu.VMEM((rows_per_w,), jnp.int32),
            pltpu.VMEM((rows_per_w,), jnp.int32),
            pltpu.VMEM((2 * SEQ, EMB_DIM), jnp.float32),
            pltpu.VMEM((2, EMB_DIM), jnp.float32),
            [pltpu.VMEM((CHUNK, EMB_DIM), jnp.float32) for _ in range(2)],
            [pltpu.VMEM((CHUNK, EMB_DIM), jnp.float32) for _ in range(2)],
            [pltpu.SemaphoreType.DMA for _ in range(2)],
            [pltpu.SemaphoreType.DMA for _ in range(2)],
        ],
    )
    out = run(idx, sidx, emb_table, seg_table, pos)
    return out.reshape(B, L, EMB_DIM)


# speedup vs baseline: 29.4984x; 2.6810x over previous
import functools

import jax
import jax.numpy as jnp
import numpy as np
from jax import lax
from jax.experimental import pallas as pl
from jax.experimental.pallas import tpu as pltpu
from jax.experimental.pallas import tpu_sc as plsc

EMB_DIM = 128
SEQ = 200
SCALE = float(np.sqrt(float(EMB_DIM)))

NUM_CORES = 2
NUM_SUBCORES = 16
NUM_WORKERS = NUM_CORES * NUM_SUBCORES
CHUNK = 64
VEC = 16
GRPS = EMB_DIM // VEC


def _pos_table(max_length, emb_dim):
    pos = np.arange(max_length)[:, np.newaxis]
    div_term = np.exp(np.arange(0, emb_dim, 2) * -(np.log(10000.0) / emb_dim))
    pos_emb = pos * div_term
    pos_emb = np.stack([np.sin(pos_emb), np.cos(pos_emb)], axis=1).reshape(max_length, -1)
    pos_emb[1:, 1::2] = 0
    return pos_emb.astype(np.float32)


_POS = _pos_table(SEQ + 1, EMB_DIM)[:SEQ]


def _encode_body(idx_hbm, sidx_hbm, emb_hbm, seg_hbm, pos_hbm, out_hbm,
                 idx_all, sidx_all, aidx_all, comb, comb_sh, segv, rows, addbuf, obuf,
                 gsem, asem, osem, rows_per_w, n_chunks):
    wid = lax.axis_index("s") * NUM_CORES + lax.axis_index("c")
    wbase = wid * rows_per_w

    pltpu.sync_copy(idx_hbm.at[pl.ds(wbase, rows_per_w)], idx_all)
    pltpu.sync_copy(sidx_hbm.at[pl.ds(wbase, rows_per_w)], sidx_all)
    pltpu.sync_copy(pos_hbm, comb.at[pl.ds(0, SEQ)])
    pltpu.sync_copy(pos_hbm, comb.at[pl.ds(SEQ, SEQ)])
    pltpu.sync_copy(seg_hbm, segv)

    @plsc.parallel_loop(0, SEQ, unroll=2)
    def _(r):
        for g in range(GRPS):
            sl = pl.ds(g * VEC, VEC)
            comb[r, sl] = comb[r, sl] + segv[0, sl]
            comb[SEQ + r, sl] = comb[SEQ + r, sl] + segv[1, sl]

    @pl.when(lax.axis_index("s") == 0)
    def _():
        pltpu.sync_copy(comb, comb_sh)

    plsc.subcore_barrier()

    @plsc.parallel_loop(0, rows_per_w // VEC, unroll=2)
    def _(v):
        base = v * VEC
        l16 = lax.rem(base + lax.iota(jnp.int32, VEC), SEQ)
        aidx_all[pl.ds(base, VEC)] = sidx_all[pl.ds(base, VEC)] * SEQ + l16

    def issue(s, ci):
        off = ci * CHUNK
        pltpu.async_copy(emb_hbm.at[idx_all.at[pl.ds(off, CHUNK)]], rows[s], gsem[s])
        pltpu.async_copy(comb_sh.at[aidx_all.at[pl.ds(off, CHUNK)]], addbuf[s], asem[s])

    def wait_gathers(s, ci):
        off = ci * CHUNK
        pltpu.make_async_copy(emb_hbm.at[idx_all.at[pl.ds(off, CHUNK)]], rows[s], gsem[s]).wait()
        pltpu.make_async_copy(comb_sh.at[aidx_all.at[pl.ds(off, CHUNK)]], addbuf[s], asem[s]).wait()

    def out_start(s, ci):
        base = wbase + ci * CHUNK
        pltpu.async_copy(obuf[s], out_hbm.at[pl.ds(base, CHUNK)], osem[s])

    def out_wait(s, ci):
        base = wbase + ci * CHUNK
        pltpu.make_async_copy(obuf[s], out_hbm.at[pl.ds(base, CHUNK)], osem[s]).wait()

    def compute(s, ci):
        @plsc.parallel_loop(0, CHUNK, unroll=2)
        def _(r):
            for g in range(GRPS):
                sl = pl.ds(g * VEC, VEC)
                obuf[s][r, sl] = rows[s][r, sl] * SCALE + addbuf[s][r, sl]

    issue(0, 0)
    issue(1, 1)

    def chunk_pair(ci2, carry):
        for s in (0, 1):
            ci = ci2 * 2 + s
            wait_gathers(s, ci)

            @pl.when(ci2 >= 1)
            def _():
                out_wait(s, ci - 2)

            compute(s, ci)

            @pl.when(ci + 2 < n_chunks)
            def _():
                issue(s, ci + 2)

            out_start(s, ci)
        return carry

    lax.fori_loop(0, n_chunks // 2, chunk_pair, 0, unroll=False)
    out_wait(0, n_chunks - 2)
    out_wait(1, n_chunks - 1)


def kernel(batch, segment_ids, emb_table, seg_table):
    B, L = batch.shape
    N = B * L
    rows_per_w = N // NUM_WORKERS
    n_chunks = rows_per_w // CHUNK

    idx = batch.reshape(N).astype(jnp.int32)
    sidx = segment_ids.reshape(N).astype(jnp.int32)
    pos = jnp.asarray(_POS)

    body = functools.partial(_encode_body, rows_per_w=rows_per_w, n_chunks=n_chunks)
    run = pl.kernel(
        body,
        out_type=jax.ShapeDtypeStruct((N, EMB_DIM), jnp.float32),
        mesh=plsc.VectorSubcoreMesh(
            core_axis_name="c", subcore_axis_name="s",
            num_cores=NUM_CORES, num_subcores=NUM_SUBCORES),
        scratch_types=[
            pltpu.VMEM((rows_per_w,), jnp.int32),
            pltpu.VMEM((rows_per_w,), jnp.int32),
            pltpu.VMEM((rows_per_w,), jnp.int32),
            pltpu.VMEM((2 * SEQ, EMB_DIM), jnp.float32),
            pltpu.VMEM_SHARED((2 * SEQ, EMB_DIM), jnp.float32),
            pltpu.VMEM((2, EMB_DIM), jnp.float32),
            [pltpu.VMEM((CHUNK, EMB_DIM), jnp.float32) for _ in range(2)],
            [pltpu.VMEM((CHUNK, EMB_DIM), jnp.float32) for _ in range(2)],
            [pltpu.VMEM((CHUNK, EMB_DIM), jnp.float32) for _ in range(2)],
            [pltpu.SemaphoreType.DMA for _ in range(2)],
            [pltpu.SemaphoreType.DMA for _ in range(2)],
            [pltpu.SemaphoreType.DMA for _ in range(2)],
        ],
    )
    out = run(idx, sidx, emb_table, seg_table, pos)
    return out.reshape(B, L, EMB_DIM)


# speedup vs baseline: 29.6592x; 1.0054x over previous
import functools

import jax
import jax.numpy as jnp
import numpy as np
from jax import lax
from jax.experimental import pallas as pl
from jax.experimental.pallas import tpu as pltpu
from jax.experimental.pallas import tpu_sc as plsc

EMB_DIM = 128
SEQ = 200
SCALE = float(np.sqrt(float(EMB_DIM)))

NUM_CORES = 2
NUM_SUBCORES = 16
NUM_WORKERS = NUM_CORES * NUM_SUBCORES
CHUNK = 64
VEC = 16
GRPS = EMB_DIM // VEC


def _pos_table(max_length, emb_dim):
    pos = np.arange(max_length)[:, np.newaxis]
    div_term = np.exp(np.arange(0, emb_dim, 2) * -(np.log(10000.0) / emb_dim))
    pos_emb = pos * div_term
    pos_emb = np.stack([np.sin(pos_emb), np.cos(pos_emb)], axis=1).reshape(max_length, -1)
    pos_emb[1:, 1::2] = 0
    return pos_emb.astype(np.float32)


_POS = _pos_table(SEQ + 1, EMB_DIM)[:SEQ]


def _encode_body(idx_hbm, sidx_hbm, emb_hbm, seg_hbm, pos_hbm, out_hbm,
                 idx_all, sidx_all, aidx_all, comb, comb_sh, segv, rows, addbuf, obuf,
                 gsem, asem, osem, rows_per_w, n_chunks):
    wid = lax.axis_index("s") * NUM_CORES + lax.axis_index("c")
    wbase = wid * rows_per_w

    pltpu.sync_copy(idx_hbm.at[pl.ds(wbase, rows_per_w)], idx_all)
    pltpu.sync_copy(sidx_hbm.at[pl.ds(wbase, rows_per_w)], sidx_all)
    pltpu.sync_copy(pos_hbm, comb.at[pl.ds(0, SEQ)])
    pltpu.sync_copy(pos_hbm, comb.at[pl.ds(SEQ, SEQ)])
    pltpu.sync_copy(seg_hbm, segv)

    @plsc.parallel_loop(0, SEQ, unroll=2)
    def _(r):
        for g in range(GRPS):
            sl = pl.ds(g * VEC, VEC)
            comb[r, sl] = comb[r, sl] + segv[0, sl]
            comb[SEQ + r, sl] = comb[SEQ + r, sl] + segv[1, sl]

    @pl.when(lax.axis_index("s") == 0)
    def _():
        pltpu.sync_copy(comb, comb_sh)

    plsc.subcore_barrier()

    @plsc.parallel_loop(0, rows_per_w // VEC, unroll=2)
    def _(v):
        base = v * VEC
        l16 = lax.rem(base + lax.iota(jnp.int32, VEC), SEQ)
        aidx_all[pl.ds(base, VEC)] = sidx_all[pl.ds(base, VEC)] * SEQ + l16

    def issue(s, ci):
        off = ci * CHUNK
        pltpu.async_copy(emb_hbm.at[idx_all.at[pl.ds(off, CHUNK)]], rows[s], gsem[s])
        pltpu.async_copy(comb_sh.at[aidx_all.at[pl.ds(off, CHUNK)]], addbuf[s], asem[s])

    def wait_gathers(s, ci):
        off = ci * CHUNK
        pltpu.make_async_copy(emb_hbm.at[idx_all.at[pl.ds(off, CHUNK)]], rows[s], gsem[s]).wait()
        pltpu.make_async_copy(comb_sh.at[aidx_all.at[pl.ds(off, CHUNK)]], addbuf[s], asem[s]).wait()

    def out_start(s, ci):
        base = wbase + ci * CHUNK
        pltpu.async_copy(obuf[s], out_hbm.at[pl.ds(base, CHUNK)], osem[s])

    def out_wait(s, ci):
        base = wbase + ci * CHUNK
        pltpu.make_async_copy(obuf[s], out_hbm.at[pl.ds(base, CHUNK)], osem[s]).wait()

    def compute(s, ci):
        @plsc.parallel_loop(0, CHUNK, unroll=4)
        def _(r):
            for g in range(GRPS):
                sl = pl.ds(g * VEC, VEC)
                obuf[s][r, sl] = rows[s][r, sl] * SCALE + addbuf[s][r, sl]

    issue(0, 0)
    issue(1, 1)

    def chunk_pair(ci2, carry):
        for s in (0, 1):
            ci = ci2 * 2 + s
            wait_gathers(s, ci)

            @pl.when(ci2 >= 1)
            def _():
                out_wait(s, ci - 2)

            compute(s, ci)

            @pl.when(ci + 2 < n_chunks)
            def _():
                issue(s, ci + 2)

            out_start(s, ci)
        return carry

    lax.fori_loop(0, n_chunks // 2, chunk_pair, 0, unroll=False)
    out_wait(0, n_chunks - 2)
    out_wait(1, n_chunks - 1)


def kernel(batch, segment_ids, emb_table, seg_table):
    B, L = batch.shape
    N = B * L
    rows_per_w = N // NUM_WORKERS
    n_chunks = rows_per_w // CHUNK

    idx = batch.reshape(N).astype(jnp.int32)
    sidx = segment_ids.reshape(N).astype(jnp.int32)
    pos = jnp.asarray(_POS)

    body = functools.partial(_encode_body, rows_per_w=rows_per_w, n_chunks=n_chunks)
    run = pl.kernel(
        body,
        out_type=jax.ShapeDtypeStruct((N, EMB_DIM), jnp.float32),
        mesh=plsc.VectorSubcoreMesh(
            core_axis_name="c", subcore_axis_name="s",
            num_cores=NUM_CORES, num_subcores=NUM_SUBCORES),
        scratch_types=[
            pltpu.VMEM((rows_per_w,), jnp.int32),
            pltpu.VMEM((rows_per_w,), jnp.int32),
            pltpu.VMEM((rows_per_w,), jnp.int32),
            pltpu.VMEM((2 * SEQ, EMB_DIM), jnp.float32),
            pltpu.VMEM_SHARED((2 * SEQ, EMB_DIM), jnp.float32),
            pltpu.VMEM((2, EMB_DIM), jnp.float32),
            [pltpu.VMEM((CHUNK, EMB_DIM), jnp.float32) for _ in range(2)],
            [pltpu.VMEM((CHUNK, EMB_DIM), jnp.float32) for _ in range(2)],
            [pltpu.VMEM((CHUNK, EMB_DIM), jnp.float32) for _ in range(2)],
            [pltpu.SemaphoreType.DMA for _ in range(2)],
            [pltpu.SemaphoreType.DMA for _ in range(2)],
            [pltpu.SemaphoreType.DMA for _ in range(2)],
        ],
    )
    out = run(idx, sidx, emb_table, seg_table, pos)
    return out.reshape(B, L, EMB_DIM)


# speedup vs baseline: 32.3554x; 1.0909x over previous
import functools

import jax
import jax.numpy as jnp
import numpy as np
from jax import lax
from jax.experimental import pallas as pl
from jax.experimental.pallas import tpu as pltpu
from jax.experimental.pallas import tpu_sc as plsc

EMB_DIM = 128
SEQ = 200
SCALE = float(np.sqrt(float(EMB_DIM)))

NUM_CORES = 2
NUM_SUBCORES = 16
NUM_WORKERS = NUM_CORES * NUM_SUBCORES
CHUNK = 64
VEC = 16
GRPS = EMB_DIM // VEC


def _pos_table(max_length, emb_dim):
    pos = np.arange(max_length)[:, np.newaxis]
    div_term = np.exp(np.arange(0, emb_dim, 2) * -(np.log(10000.0) / emb_dim))
    pos_emb = pos * div_term
    pos_emb = np.stack([np.sin(pos_emb), np.cos(pos_emb)], axis=1).reshape(max_length, -1)
    pos_emb[1:, 1::2] = 0
    return pos_emb.astype(np.float32)


_POS = _pos_table(SEQ + 1, EMB_DIM)[:SEQ]
PSEQ = 256
_POSP = np.concatenate([_POS, np.zeros((PSEQ - SEQ, EMB_DIM), np.float32)], axis=0)


def _encode_body(idx_hbm, sidx_hbm, emb_hbm, seg_hbm, pos_hbm, out_hbm,
                 idx_all, sidx_all, aidx_all, comb_sh, segv, rows, addbuf, obuf,
                 gsem, asem, osem, rows_per_w, n_chunks):
    wid = lax.axis_index("s") * NUM_CORES + lax.axis_index("c")
    wbase = wid * rows_per_w

    pltpu.sync_copy(idx_hbm.at[pl.ds(wbase, rows_per_w)], idx_all)
    pltpu.sync_copy(sidx_hbm.at[pl.ds(wbase, rows_per_w)], sidx_all)
    pltpu.sync_copy(seg_hbm, segv)

    tid = lax.axis_index("s")
    BUILD = 2 * PSEQ // NUM_SUBCORES
    half = tid // (NUM_SUBCORES // 2)
    l0 = pl.multiple_of(lax.rem(tid, NUM_SUBCORES // 2) * BUILD, 8)
    pltpu.sync_copy(pos_hbm.at[pl.ds(l0, BUILD)], rows[0].at[pl.ds(0, BUILD)])

    @plsc.parallel_loop(0, BUILD)
    def _(r):
        for g in range(GRPS):
            sl = pl.ds(g * VEC, VEC)
            rows[0][r, sl] = rows[0][r, sl] + segv[half, sl]

    pltpu.sync_copy(rows[0].at[pl.ds(0, BUILD)],
                    comb_sh.at[pl.ds(pl.multiple_of(tid * BUILD, 8), BUILD)])
    plsc.subcore_barrier()

    @plsc.parallel_loop(0, rows_per_w // VEC, unroll=2)
    def _(v):
        base = v * VEC
        l16 = lax.rem(base + lax.iota(jnp.int32, VEC), SEQ)
        aidx_all[pl.ds(base, VEC)] = sidx_all[pl.ds(base, VEC)] * PSEQ + l16

    def issue(s, ci):
        off = ci * CHUNK
        pltpu.async_copy(emb_hbm.at[idx_all.at[pl.ds(off, CHUNK)]], rows[s], gsem[s])
        pltpu.async_copy(comb_sh.at[aidx_all.at[pl.ds(off, CHUNK)]], addbuf[s], asem[s])

    def wait_gathers(s, ci):
        off = ci * CHUNK
        pltpu.make_async_copy(emb_hbm.at[idx_all.at[pl.ds(off, CHUNK)]], rows[s], gsem[s]).wait()
        pltpu.make_async_copy(comb_sh.at[aidx_all.at[pl.ds(off, CHUNK)]], addbuf[s], asem[s]).wait()

    def out_start(s, ci):
        base = wbase + ci * CHUNK
        pltpu.async_copy(obuf[s], out_hbm.at[pl.ds(base, CHUNK)], osem[s])

    def out_wait(s, ci):
        base = wbase + ci * CHUNK
        pltpu.make_async_copy(obuf[s], out_hbm.at[pl.ds(base, CHUNK)], osem[s]).wait()

    def compute(s, ci):
        @plsc.parallel_loop(0, CHUNK, unroll=4)
        def _(r):
            for g in range(GRPS):
                sl = pl.ds(g * VEC, VEC)
                obuf[s][r, sl] = rows[s][r, sl] * SCALE + addbuf[s][r, sl]

    issue(0, 0)
    issue(1, 1)

    def chunk_pair(ci2, carry):
        for s in (0, 1):
            ci = ci2 * 2 + s
            wait_gathers(s, ci)

            @pl.when(ci2 >= 1)
            def _():
                out_wait(s, ci - 2)

            compute(s, ci)

            @pl.when(ci + 2 < n_chunks)
            def _():
                issue(s, ci + 2)

            out_start(s, ci)
        return carry

    lax.fori_loop(0, n_chunks // 2, chunk_pair, 0, unroll=False)
    out_wait(0, n_chunks - 2)
    out_wait(1, n_chunks - 1)


def kernel(batch, segment_ids, emb_table, seg_table):
    B, L = batch.shape
    N = B * L
    rows_per_w = N // NUM_WORKERS
    n_chunks = rows_per_w // CHUNK

    idx = batch.reshape(N).astype(jnp.int32)
    sidx = segment_ids.reshape(N).astype(jnp.int32)
    pos = jnp.asarray(_POSP)

    body = functools.partial(_encode_body, rows_per_w=rows_per_w, n_chunks=n_chunks)
    run = pl.kernel(
        body,
        out_type=jax.ShapeDtypeStruct((N, EMB_DIM), jnp.float32),
        mesh=plsc.VectorSubcoreMesh(
            core_axis_name="c", subcore_axis_name="s",
            num_cores=NUM_CORES, num_subcores=NUM_SUBCORES),
        scratch_types=[
            pltpu.VMEM((rows_per_w,), jnp.int32),
            pltpu.VMEM((rows_per_w,), jnp.int32),
            pltpu.VMEM((rows_per_w,), jnp.int32),
            pltpu.VMEM_SHARED((2 * PSEQ, EMB_DIM), jnp.float32),
            pltpu.VMEM((2, EMB_DIM), jnp.float32),
            [pltpu.VMEM((CHUNK, EMB_DIM), jnp.float32) for _ in range(2)],
            [pltpu.VMEM((CHUNK, EMB_DIM), jnp.float32) for _ in range(2)],
            [pltpu.VMEM((CHUNK, EMB_DIM), jnp.float32) for _ in range(2)],
            [pltpu.SemaphoreType.DMA for _ in range(2)],
            [pltpu.SemaphoreType.DMA for _ in range(2)],
            [pltpu.SemaphoreType.DMA for _ in range(2)],
        ],
    )
    out = run(idx, sidx, emb_table, seg_table, pos)
    return out.reshape(B, L, EMB_DIM)


# speedup vs baseline: 37.1709x; 1.1488x over previous
import functools

import jax
import jax.numpy as jnp
import numpy as np
from jax import lax
from jax.experimental import pallas as pl
from jax.experimental.pallas import tpu as pltpu
from jax.experimental.pallas import tpu_sc as plsc

EMB_DIM = 128
SEQ = 200
SCALE = float(np.sqrt(float(EMB_DIM)))

NUM_CORES = 2
NUM_SUBCORES = 16
NUM_WORKERS = NUM_CORES * NUM_SUBCORES
CHUNK = 128
VEC = 16
GRPS = EMB_DIM // VEC


def _pos_table(max_length, emb_dim):
    pos = np.arange(max_length)[:, np.newaxis]
    div_term = np.exp(np.arange(0, emb_dim, 2) * -(np.log(10000.0) / emb_dim))
    pos_emb = pos * div_term
    pos_emb = np.stack([np.sin(pos_emb), np.cos(pos_emb)], axis=1).reshape(max_length, -1)
    pos_emb[1:, 1::2] = 0
    return pos_emb.astype(np.float32)


_POS = _pos_table(SEQ + 1, EMB_DIM)[:SEQ]
PSEQ = 256
_POSP = np.concatenate([_POS, np.zeros((PSEQ - SEQ, EMB_DIM), np.float32)], axis=0)


def _encode_body(idx_hbm, sidx_hbm, emb_hbm, seg_hbm, pos_hbm, out_hbm,
                 idx_all, sidx_all, aidx_all, comb_sh, segv, rows, addbuf, obuf,
                 gsem, asem, osem, rows_per_w, n_chunks):
    wid = lax.axis_index("s") * NUM_CORES + lax.axis_index("c")
    wbase = wid * rows_per_w

    pltpu.sync_copy(idx_hbm.at[pl.ds(wbase, rows_per_w)], idx_all)
    pltpu.sync_copy(sidx_hbm.at[pl.ds(wbase, rows_per_w)], sidx_all)
    pltpu.sync_copy(seg_hbm, segv)

    tid = lax.axis_index("s")
    BUILD = 2 * PSEQ // NUM_SUBCORES
    half = tid // (NUM_SUBCORES // 2)
    l0 = pl.multiple_of(lax.rem(tid, NUM_SUBCORES // 2) * BUILD, 8)
    pltpu.sync_copy(pos_hbm.at[pl.ds(l0, BUILD)], rows[0].at[pl.ds(0, BUILD)])

    @plsc.parallel_loop(0, BUILD)
    def _(r):
        for g in range(GRPS):
            sl = pl.ds(g * VEC, VEC)
            rows[0][r, sl] = rows[0][r, sl] + segv[half, sl]

    pltpu.sync_copy(rows[0].at[pl.ds(0, BUILD)],
                    comb_sh.at[pl.ds(pl.multiple_of(tid * BUILD, 8), BUILD)])
    plsc.subcore_barrier()

    @plsc.parallel_loop(0, rows_per_w // VEC, unroll=2)
    def _(v):
        base = v * VEC
        l16 = lax.rem(base + lax.iota(jnp.int32, VEC), SEQ)
        aidx_all[pl.ds(base, VEC)] = sidx_all[pl.ds(base, VEC)] * PSEQ + l16

    def issue(s, ci):
        off = ci * CHUNK
        pltpu.async_copy(emb_hbm.at[idx_all.at[pl.ds(off, CHUNK)]], rows[s], gsem[s])
        pltpu.async_copy(comb_sh.at[aidx_all.at[pl.ds(off, CHUNK)]], addbuf[s], asem[s])

    def wait_gathers(s, ci):
        off = ci * CHUNK
        pltpu.make_async_copy(emb_hbm.at[idx_all.at[pl.ds(off, CHUNK)]], rows[s], gsem[s]).wait()
        pltpu.make_async_copy(comb_sh.at[aidx_all.at[pl.ds(off, CHUNK)]], addbuf[s], asem[s]).wait()

    def out_start(s, ci):
        base = wbase + ci * CHUNK
        pltpu.async_copy(obuf[s], out_hbm.at[pl.ds(base, CHUNK)], osem[s])

    def out_wait(s, ci):
        base = wbase + ci * CHUNK
        pltpu.make_async_copy(obuf[s], out_hbm.at[pl.ds(base, CHUNK)], osem[s]).wait()

    def compute(s, ci):
        @plsc.parallel_loop(0, CHUNK, unroll=4)
        def _(r):
            for g in range(GRPS):
                sl = pl.ds(g * VEC, VEC)
                obuf[s][r, sl] = rows[s][r, sl] * SCALE + addbuf[s][r, sl]

    issue(0, 0)
    issue(1, 1)

    def chunk_pair(ci2, carry):
        for s in (0, 1):
            ci = ci2 * 2 + s
            wait_gathers(s, ci)

            @pl.when(ci2 >= 1)
            def _():
                out_wait(s, ci - 2)

            compute(s, ci)

            @pl.when(ci + 2 < n_chunks)
            def _():
                issue(s, ci + 2)

            out_start(s, ci)
        return carry

    lax.fori_loop(0, n_chunks // 2, chunk_pair, 0, unroll=False)
    out_wait(0, n_chunks - 2)
    out_wait(1, n_chunks - 1)


def kernel(batch, segment_ids, emb_table, seg_table):
    B, L = batch.shape
    N = B * L
    rows_per_w = N // NUM_WORKERS
    n_chunks = rows_per_w // CHUNK

    idx = batch.reshape(N).astype(jnp.int32)
    sidx = segment_ids.reshape(N).astype(jnp.int32)
    pos = jnp.asarray(_POSP)

    body = functools.partial(_encode_body, rows_per_w=rows_per_w, n_chunks=n_chunks)
    run = pl.kernel(
        body,
        out_type=jax.ShapeDtypeStruct((N, EMB_DIM), jnp.float32),
        mesh=plsc.VectorSubcoreMesh(
            core_axis_name="c", subcore_axis_name="s",
            num_cores=NUM_CORES, num_subcores=NUM_SUBCORES),
        scratch_types=[
            pltpu.VMEM((rows_per_w,), jnp.int32),
            pltpu.VMEM((rows_per_w,), jnp.int32),
            pltpu.VMEM((rows_per_w,), jnp.int32),
            pltpu.VMEM_SHARED((2 * PSEQ, EMB_DIM), jnp.float32),
            pltpu.VMEM((2, EMB_DIM), jnp.float32),
            [pltpu.VMEM((CHUNK, EMB_DIM), jnp.float32) for _ in range(2)],
            [pltpu.VMEM((CHUNK, EMB_DIM), jnp.float32) for _ in range(2)],
            [pltpu.VMEM((CHUNK, EMB_DIM), jnp.float32) for _ in range(2)],
            [pltpu.SemaphoreType.DMA for _ in range(2)],
            [pltpu.SemaphoreType.DMA for _ in range(2)],
            [pltpu.SemaphoreType.DMA for _ in range(2)],
        ],
    )
    out = run(idx, sidx, emb_table, seg_table, pos)
    return out.reshape(B, L, EMB_DIM)


# speedup vs baseline: 38.0067x; 1.0225x over previous
import functools

import jax
import jax.numpy as jnp
import numpy as np
from jax import lax
from jax.experimental import pallas as pl
from jax.experimental.pallas import tpu as pltpu
from jax.experimental.pallas import tpu_sc as plsc

EMB_DIM = 128
SEQ = 200
SCALE = float(np.sqrt(float(EMB_DIM)))

NUM_CORES = 2
NUM_SUBCORES = 16
NUM_WORKERS = NUM_CORES * NUM_SUBCORES
CHUNK = 128
VEC = 16
GRPS = EMB_DIM // VEC


def _pos_table(max_length, emb_dim):
    pos = np.arange(max_length)[:, np.newaxis]
    div_term = np.exp(np.arange(0, emb_dim, 2) * -(np.log(10000.0) / emb_dim))
    pos_emb = pos * div_term
    pos_emb = np.stack([np.sin(pos_emb), np.cos(pos_emb)], axis=1).reshape(max_length, -1)
    pos_emb[1:, 1::2] = 0
    return pos_emb.astype(np.float32)


_POS = _pos_table(SEQ + 1, EMB_DIM)[:SEQ]
PSEQ = 256
_POSP = np.concatenate([_POS, np.zeros((PSEQ - SEQ, EMB_DIM), np.float32)], axis=0)


def _encode_body(idx_hbm, sidx_hbm, emb_hbm, seg_hbm, pos_hbm, out_hbm,
                 idx_all, sidx_all, aidx_all, comb_sh, segv, rows, addbuf, obuf,
                 gsem, gsemb, asem, osem, rows_per_w, n_chunks):
    wid = lax.axis_index("s") * NUM_CORES + lax.axis_index("c")
    wbase = wid * rows_per_w

    pltpu.sync_copy(idx_hbm.at[pl.ds(wbase, rows_per_w)], idx_all)
    pltpu.sync_copy(sidx_hbm.at[pl.ds(wbase, rows_per_w)], sidx_all)
    pltpu.sync_copy(seg_hbm, segv)

    tid = lax.axis_index("s")
    BUILD = 2 * PSEQ // NUM_SUBCORES
    half = tid // (NUM_SUBCORES // 2)
    l0 = pl.multiple_of(lax.rem(tid, NUM_SUBCORES // 2) * BUILD, 8)
    pltpu.sync_copy(pos_hbm.at[pl.ds(l0, BUILD)], rows[0].at[pl.ds(0, BUILD)])

    @plsc.parallel_loop(0, BUILD)
    def _(r):
        for g in range(GRPS):
            sl = pl.ds(g * VEC, VEC)
            rows[0][r, sl] = rows[0][r, sl] + segv[half, sl]

    pltpu.sync_copy(rows[0].at[pl.ds(0, BUILD)],
                    comb_sh.at[pl.ds(pl.multiple_of(tid * BUILD, 8), BUILD)])
    plsc.subcore_barrier()

    @plsc.parallel_loop(0, rows_per_w // VEC, unroll=2)
    def _(v):
        base = v * VEC
        l16 = lax.rem(base + lax.iota(jnp.int32, VEC), SEQ)
        aidx_all[pl.ds(base, VEC)] = sidx_all[pl.ds(base, VEC)] * PSEQ + l16

    HCH = CHUNK // 2

    def issue(s, ci):
        off = ci * CHUNK
        pltpu.async_copy(emb_hbm.at[idx_all.at[pl.ds(off, HCH)]],
                         rows[s].at[pl.ds(0, HCH)], gsem[s])
        pltpu.async_copy(emb_hbm.at[idx_all.at[pl.ds(off + HCH, HCH)]],
                         rows[s].at[pl.ds(HCH, HCH)], gsemb[s])
        pltpu.async_copy(comb_sh.at[aidx_all.at[pl.ds(off, CHUNK)]], addbuf[s], asem[s])

    def wait_gathers(s, ci):
        off = ci * CHUNK
        pltpu.make_async_copy(emb_hbm.at[idx_all.at[pl.ds(off, HCH)]],
                              rows[s].at[pl.ds(0, HCH)], gsem[s]).wait()
        pltpu.make_async_copy(emb_hbm.at[idx_all.at[pl.ds(off + HCH, HCH)]],
                              rows[s].at[pl.ds(HCH, HCH)], gsemb[s]).wait()
        pltpu.make_async_copy(comb_sh.at[aidx_all.at[pl.ds(off, CHUNK)]], addbuf[s], asem[s]).wait()

    def out_start(s, ci):
        base = wbase + ci * CHUNK
        pltpu.async_copy(obuf[s], out_hbm.at[pl.ds(base, CHUNK)], osem[s])

    def out_wait(s, ci):
        base = wbase + ci * CHUNK
        pltpu.make_async_copy(obuf[s], out_hbm.at[pl.ds(base, CHUNK)], osem[s]).wait()

    def compute(s, ci):
        @plsc.parallel_loop(0, CHUNK, unroll=4)
        def _(r):
            for g in range(GRPS):
                sl = pl.ds(g * VEC, VEC)
                obuf[s][r, sl] = rows[s][r, sl] * SCALE + addbuf[s][r, sl]

    issue(0, 0)
    issue(1, 1)

    def chunk_pair(ci2, carry):
        for s in (0, 1):
            ci = ci2 * 2 + s
            wait_gathers(s, ci)

            @pl.when(ci2 >= 1)
            def _():
                out_wait(s, ci - 2)

            compute(s, ci)

            @pl.when(ci + 2 < n_chunks)
            def _():
                issue(s, ci + 2)

            out_start(s, ci)
        return carry

    lax.fori_loop(0, n_chunks // 2, chunk_pair, 0, unroll=False)
    out_wait(0, n_chunks - 2)
    out_wait(1, n_chunks - 1)


def kernel(batch, segment_ids, emb_table, seg_table):
    B, L = batch.shape
    N = B * L
    rows_per_w = N // NUM_WORKERS
    n_chunks = rows_per_w // CHUNK

    idx = batch.reshape(N).astype(jnp.int32)
    sidx = segment_ids.reshape(N).astype(jnp.int32)
    pos = jnp.asarray(_POSP)

    body = functools.partial(_encode_body, rows_per_w=rows_per_w, n_chunks=n_chunks)
    run = pl.kernel(
        body,
        out_type=jax.ShapeDtypeStruct((N, EMB_DIM), jnp.float32),
        mesh=plsc.VectorSubcoreMesh(
            core_axis_name="c", subcore_axis_name="s",
            num_cores=NUM_CORES, num_subcores=NUM_SUBCORES),
        scratch_types=[
            pltpu.VMEM((rows_per_w,), jnp.int32),
            pltpu.VMEM((rows_per_w,), jnp.int32),
            pltpu.VMEM((rows_per_w,), jnp.int32),
            pltpu.VMEM_SHARED((2 * PSEQ, EMB_DIM), jnp.float32),
            pltpu.VMEM((2, EMB_DIM), jnp.float32),
            [pltpu.VMEM((CHUNK, EMB_DIM), jnp.float32) for _ in range(2)],
            [pltpu.VMEM((CHUNK, EMB_DIM), jnp.float32) for _ in range(2)],
            [pltpu.VMEM((CHUNK, EMB_DIM), jnp.float32) for _ in range(2)],
            [pltpu.SemaphoreType.DMA for _ in range(2)],
            [pltpu.SemaphoreType.DMA for _ in range(2)],
            [pltpu.SemaphoreType.DMA for _ in range(2)],
            [pltpu.SemaphoreType.DMA for _ in range(2)],
        ],
    )
    out = run(idx, sidx, emb_table, seg_table, pos)
    return out.reshape(B, L, EMB_DIM)


# speedup vs baseline: 38.0344x; 1.0007x over previous
import functools

import jax
import jax.numpy as jnp
import numpy as np
from jax import lax
from jax.experimental import pallas as pl
from jax.experimental.pallas import tpu as pltpu
from jax.experimental.pallas import tpu_sc as plsc

EMB_DIM = 128
SEQ = 200
SCALE = float(np.sqrt(float(EMB_DIM)))

NUM_CORES = 2
NUM_SUBCORES = 16
NUM_WORKERS = NUM_CORES * NUM_SUBCORES
CHUNK = 128
VEC = 16
GRPS = EMB_DIM // VEC


def _pos_table(max_length, emb_dim):
    pos = np.arange(max_length)[:, np.newaxis]
    div_term = np.exp(np.arange(0, emb_dim, 2) * -(np.log(10000.0) / emb_dim))
    pos_emb = pos * div_term
    pos_emb = np.stack([np.sin(pos_emb), np.cos(pos_emb)], axis=1).reshape(max_length, -1)
    pos_emb[1:, 1::2] = 0
    return pos_emb.astype(np.float32)


_POS = _pos_table(SEQ + 1, EMB_DIM)[:SEQ]
PSEQ = 256
_POSP = np.concatenate([_POS, np.zeros((PSEQ - SEQ, EMB_DIM), np.float32)], axis=0)


def _encode_body(idx_hbm, sidx_hbm, emb_hbm, seg_hbm, pos_hbm, out_hbm,
                 idx_all, sidx_all, aidx_all, comb_sh, segv, rows, addbuf, obuf,
                 gsem, gsemb, gsemc, gsemd, asem, osem, rows_per_w, n_chunks):
    wid = lax.axis_index("s") * NUM_CORES + lax.axis_index("c")
    wbase = wid * rows_per_w

    pltpu.sync_copy(idx_hbm.at[pl.ds(wbase, rows_per_w)], idx_all)
    pltpu.sync_copy(sidx_hbm.at[pl.ds(wbase, rows_per_w)], sidx_all)
    pltpu.sync_copy(seg_hbm, segv)

    tid = lax.axis_index("s")
    BUILD = 2 * PSEQ // NUM_SUBCORES
    half = tid // (NUM_SUBCORES // 2)
    l0 = pl.multiple_of(lax.rem(tid, NUM_SUBCORES // 2) * BUILD, 8)
    pltpu.sync_copy(pos_hbm.at[pl.ds(l0, BUILD)], rows[0].at[pl.ds(0, BUILD)])

    @plsc.parallel_loop(0, BUILD)
    def _(r):
        for g in range(GRPS):
            sl = pl.ds(g * VEC, VEC)
            rows[0][r, sl] = rows[0][r, sl] + segv[half, sl]

    pltpu.sync_copy(rows[0].at[pl.ds(0, BUILD)],
                    comb_sh.at[pl.ds(pl.multiple_of(tid * BUILD, 8), BUILD)])
    plsc.subcore_barrier()

    @plsc.parallel_loop(0, rows_per_w // VEC, unroll=2)
    def _(v):
        base = v * VEC
        l16 = lax.rem(base + lax.iota(jnp.int32, VEC), SEQ)
        aidx_all[pl.ds(base, VEC)] = sidx_all[pl.ds(base, VEC)] * PSEQ + l16

    NSTR = 4
    HCH = CHUNK // NSTR
    gsems = [gsem, gsemb, gsemc, gsemd]

    def issue(s, ci):
        off = ci * CHUNK
        for k in range(NSTR):
            pltpu.async_copy(emb_hbm.at[idx_all.at[pl.ds(off + k * HCH, HCH)]],
                             rows[s].at[pl.ds(k * HCH, HCH)], gsems[k][s])
        pltpu.async_copy(comb_sh.at[aidx_all.at[pl.ds(off, CHUNK)]], addbuf[s], asem[s])

    def wait_gathers(s, ci):
        off = ci * CHUNK
        for k in range(NSTR):
            pltpu.make_async_copy(emb_hbm.at[idx_all.at[pl.ds(off + k * HCH, HCH)]],
                                  rows[s].at[pl.ds(k * HCH, HCH)], gsems[k][s]).wait()
        pltpu.make_async_copy(comb_sh.at[aidx_all.at[pl.ds(off, CHUNK)]], addbuf[s], asem[s]).wait()

    def out_start(s, ci):
        base = wbase + ci * CHUNK
        pltpu.async_copy(obuf[s], out_hbm.at[pl.ds(base, CHUNK)], osem[s])

    def out_wait(s, ci):
        base = wbase + ci * CHUNK
        pltpu.make_async_copy(obuf[s], out_hbm.at[pl.ds(base, CHUNK)], osem[s]).wait()

    def compute(s, ci):
        @plsc.parallel_loop(0, CHUNK, unroll=4)
        def _(r):
            for g in range(GRPS):
                sl = pl.ds(g * VEC, VEC)
                obuf[s][r, sl] = rows[s][r, sl] * SCALE + addbuf[s][r, sl]

    issue(0, 0)
    issue(1, 1)

    def chunk_pair(ci2, carry):
        for s in (0, 1):
            ci = ci2 * 2 + s
            wait_gathers(s, ci)

            @pl.when(ci2 >= 1)
            def _():
                out_wait(s, ci - 2)

            compute(s, ci)

            @pl.when(ci + 2 < n_chunks)
            def _():
                issue(s, ci + 2)

            out_start(s, ci)
        return carry

    lax.fori_loop(0, n_chunks // 2, chunk_pair, 0, unroll=False)
    out_wait(0, n_chunks - 2)
    out_wait(1, n_chunks - 1)


def kernel(batch, segment_ids, emb_table, seg_table):
    B, L = batch.shape
    N = B * L
    rows_per_w = N // NUM_WORKERS
    n_chunks = rows_per_w // CHUNK

    idx = batch.reshape(N).astype(jnp.int32)
    sidx = segment_ids.reshape(N).astype(jnp.int32)
    pos = jnp.asarray(_POSP)

    body = functools.partial(_encode_body, rows_per_w=rows_per_w, n_chunks=n_chunks)
    run = pl.kernel(
        body,
        out_type=jax.ShapeDtypeStruct((N, EMB_DIM), jnp.float32),
        mesh=plsc.VectorSubcoreMesh(
            core_axis_name="c", subcore_axis_name="s",
            num_cores=NUM_CORES, num_subcores=NUM_SUBCORES),
        scratch_types=[
            pltpu.VMEM((rows_per_w,), jnp.int32),
            pltpu.VMEM((rows_per_w,), jnp.int32),
            pltpu.VMEM((rows_per_w,), jnp.int32),
            pltpu.VMEM_SHARED((2 * PSEQ, EMB_DIM), jnp.float32),
            pltpu.VMEM((2, EMB_DIM), jnp.float32),
            [pltpu.VMEM((CHUNK, EMB_DIM), jnp.float32) for _ in range(2)],
            [pltpu.VMEM((CHUNK, EMB_DIM), jnp.float32) for _ in range(2)],
            [pltpu.VMEM((CHUNK, EMB_DIM), jnp.float32) for _ in range(2)],
            [pltpu.SemaphoreType.DMA for _ in range(2)],
            [pltpu.SemaphoreType.DMA for _ in range(2)],
            [pltpu.SemaphoreType.DMA for _ in range(2)],
            [pltpu.SemaphoreType.DMA for _ in range(2)],
            [pltpu.SemaphoreType.DMA for _ in range(2)],
            [pltpu.SemaphoreType.DMA for _ in range(2)],
        ],
    )
    out = run(idx, sidx, emb_table, seg_table, pos)
    return out.reshape(B, L, EMB_DIM)
